# Initial kernel scaffold; baseline (speedup 1.0000x reference)
import jax
import jax.numpy as jnp
from jax.experimental import pallas as pl


def kernel(v, batch, weight):
    raise NotImplementedError("write your pallas kernel here")


# speedup vs baseline: 3.3834x; 3.3834x over previous
import jax
import jax.numpy as jnp
from jax import lax
from jax.experimental import pallas as pl
from jax.experimental.pallas import tpu as pltpu
from jax.experimental.pallas import tpu_sc as plsc

N_ATOMS = 100000
HIDDEN = 128
N_MOL = 5000
M_PAD = 5120
PLANE_STRIDE = 8192

NC = 2
NS = 16
N_TILES = NC * NS
N_PAD = 131072
ATOMS_PER_TILE = N_PAD // N_TILES
CHUNKS_PER_TILE = ATOMS_PER_TILE // 128

TC_BLOCK = 4000


def _tc_body(v_ref, w_ref, mu_ref):
    vb = v_ref[...].reshape(3 * TC_BLOCK, HIDDEN)
    w = w_ref[...]
    r = lax.dot_general(w, vb, (((1,), (1,)), ((), ())),
                        preferred_element_type=jnp.float32)
    mu_ref[pl.program_id(0), :] = r[0]


def _tc_project(vt, w2d):
    grid = N_ATOMS // TC_BLOCK
    return pl.pallas_call(
        _tc_body,
        grid=(grid,),
        in_specs=[
            pl.BlockSpec((3, TC_BLOCK, HIDDEN), lambda i: (0, i, 0)),
            pl.BlockSpec((1, HIDDEN), lambda i: (0, 0)),
        ],
        out_specs=pl.BlockSpec((grid, 3 * TC_BLOCK), lambda i: (0, 0)),
        out_shape=jax.ShapeDtypeStruct((grid, 3 * TC_BLOCK), jnp.float32),
    )(vt, w2d)


def _sc_body(mu0_hbm, mu1_hbm, mu2_hbm, idx_hbm, zeros_hbm, out_hbm,
             idx_v, v0, v1, v2, a0, a1, a2):
    c = lax.axis_index("c")
    s = lax.axis_index("s")
    wid = s * NC + c

    @pl.when(s == 0)
    def _zero():
        pltpu.sync_copy(zeros_hbm, a0)
        pltpu.sync_copy(zeros_hbm, a1)
        pltpu.sync_copy(zeros_hbm, a2)

    plsc.subcore_barrier()

    base = wid * ATOMS_PER_TILE
    pltpu.sync_copy(idx_hbm.at[pl.ds(wid * CHUNKS_PER_TILE, CHUNKS_PER_TILE)], idx_v)
    pltpu.sync_copy(mu0_hbm.at[pl.ds(base, ATOMS_PER_TILE)], v0)
    pltpu.sync_copy(mu1_hbm.at[pl.ds(base, ATOMS_PER_TILE)], v1)
    pltpu.sync_copy(mu2_hbm.at[pl.ds(base, ATOMS_PER_TILE)], v2)

    def _chunk(j):
        idx = idx_v.at[j]
        pltpu.sync_copy(v0.at[pl.ds(j * 128, 128)], a0.at[idx], add=True)
        pltpu.sync_copy(v1.at[pl.ds(j * 128, 128)], a1.at[idx], add=True)
        pltpu.sync_copy(v2.at[pl.ds(j * 128, 128)], a2.at[idx], add=True)

    pl.loop(0, CHUNKS_PER_TILE)(_chunk)

    plsc.subcore_barrier()

    @pl.when(s == 0)
    def _writeout():
        pltpu.sync_copy(a0, out_hbm.at[c, pl.ds(0 * PLANE_STRIDE, M_PAD)])
        pltpu.sync_copy(a1, out_hbm.at[c, pl.ds(1 * PLANE_STRIDE, M_PAD)])
        pltpu.sync_copy(a2, out_hbm.at[c, pl.ds(2 * PLANE_STRIDE, M_PAD)])


def _sc_scatter(mu0, mu1, mu2, idx2d, zeros):
    mesh = plsc.VectorSubcoreMesh(core_axis_name="c", subcore_axis_name="s")
    fn = pl.kernel(
        _sc_body,
        out_type=jax.ShapeDtypeStruct((NC, 3 * PLANE_STRIDE), jnp.float32),
        mesh=mesh,
        scratch_types=[
            pltpu.VMEM((CHUNKS_PER_TILE, 128), jnp.int32),
            pltpu.VMEM((ATOMS_PER_TILE,), jnp.float32),
            pltpu.VMEM((ATOMS_PER_TILE,), jnp.float32),
            pltpu.VMEM((ATOMS_PER_TILE,), jnp.float32),
            pltpu.VMEM_SHARED((M_PAD,), jnp.float32),
            pltpu.VMEM_SHARED((M_PAD,), jnp.float32),
            pltpu.VMEM_SHARED((M_PAD,), jnp.float32),
        ],
    )
    return fn(mu0, mu1, mu2, idx2d, zeros)


def kernel(v, batch, weight):
    vt = jnp.transpose(v, (2, 0, 1))
    grid = N_ATOMS // TC_BLOCK
    mu_blk = _tc_project(vt, weight.reshape(1, HIDDEN))
    mu = (mu_blk.reshape(grid, 3, TC_BLOCK)
          .transpose(1, 0, 2).reshape(3, N_ATOMS))

    mu_pad = jnp.pad(mu, ((0, 0), (0, N_PAD - N_ATOMS)))
    idx2d = jnp.concatenate(
        [batch.astype(jnp.int32), jnp.full((N_PAD - N_ATOMS,), N_MOL - 1, jnp.int32)]
    ).reshape(N_PAD // 128, 128)
    zeros = jnp.zeros((M_PAD,), jnp.float32)

    partial = _sc_scatter(mu_pad[0], mu_pad[1], mu_pad[2], idx2d, zeros)
    planes = partial.reshape(NC, 3, PLANE_STRIDE)[:, :, :N_MOL]
    return (planes[0] + planes[1]).T
